# baseline (speedup 1.0000x reference)
import jax
import jax.numpy as jnp
from jax import lax
from jax.experimental import pallas as pl
from jax.experimental.pallas import tpu as pltpu
from jax.experimental.pallas import tpu_sc as plsc

C = 26
V = 100001
CLS = 28
ID = 4
DM = 32
B = 16384

NC = 2
NS = 16
NW = NC * NS
SEG = 128
NSEG = 4
CHUNK = SEG * NSEG
P = B // CHUNK
T = C * P
TPW = T // NW
G = 16


def _body(xt_hbm, tab_hbm, out_hbm, idx_v, rows_v, blk_v, sem):
    wid = lax.axis_index("s") * NC + lax.axis_index("c")
    lanes = lax.iota(jnp.int32, 16)

    def task(j, _):
        t = wid * TPW + j
        c = t // P
        b0 = (t % P) * CHUNK
        pltpu.sync_copy(xt_hbm.at[pl.ds(t * NSEG, NSEG)], idx_v)

        def group(g, _):
            iv = idx_v[g // (SEG // G), pl.ds((g % (SEG // G)) * G, G)]
            av = (iv // 8) * 8
            copies = []
            for k in range(G):
                i_al = pl.multiple_of(av[k], 8)
                copies.append(
                    pltpu.async_copy(
                        tab_hbm.at[c, pl.ds(i_al, 8)], blk_v.at[k], sem
                    )
                )
            for cp in copies:
                cp.wait()
            rvec = iv % 8
            rowb = g * G + lanes
            for col in range(CLS):
                cvec = lanes * 0 + col
                vals = plsc.load_gather(blk_v, [lanes, rvec, cvec])
                plsc.store_scatter(rows_v, [rowb, cvec], vals)
            return 0

        lax.fori_loop(0, CHUNK // G, group, 0)
        pltpu.sync_copy(rows_v, out_hbm.at[c, pl.ds(b0, CHUNK)])
        return 0

    lax.fori_loop(0, TPW, task, 0)


def kernel(x_cat, col_id, tables):
    xt = x_cat.T.reshape(C * B // SEG, SEG)

    mesh = plsc.VectorSubcoreMesh(core_axis_name="c", subcore_axis_name="s")
    f = pl.kernel(
        _body,
        out_type=jax.ShapeDtypeStruct((C, B, CLS), jnp.float32),
        mesh=mesh,
        scratch_types=[
            pltpu.VMEM((NSEG, SEG), jnp.int32),
            pltpu.VMEM((CHUNK, CLS), jnp.float32),
            pltpu.VMEM((G, 8, CLS), jnp.float32),
            pltpu.SemaphoreType.DMA,
        ],
        compiler_params=pltpu.CompilerParams(needs_layout_passes=False),
    )
    cls_t = f(xt, tables)
    cid = jnp.broadcast_to(col_id[None, :, :], (B, C, ID))
    return jnp.concatenate([cid, cls_t.transpose(1, 0, 2)], axis=-1)
